# baseline (speedup 1.0000x reference)
import functools

import jax
import jax.numpy as jnp
from jax import lax
from jax.experimental import pallas as pl
from jax.experimental.pallas import tpu as pltpu
from jax.experimental.pallas import tpu_sc as plsc

V = 100000
D = 128
B = 4096 * 200
NC = 2
NS = 16
NW = NC * NS
BPW = B // NW
CH = 128
NBUF = 5
LA = 2
NCH = BPW // CH

assert NCH % NBUF == 0

_mesh = plsc.VectorSubcoreMesh(core_axis_name="c", subcore_axis_name="s")


@functools.partial(
    pl.kernel,
    mesh=_mesh,
    out_type=jax.ShapeDtypeStruct((B, D), jnp.float32),
    scratch_types=(
        [pltpu.VMEM((BPW,), jnp.int32)]
        + [pltpu.VMEM((CH, D), jnp.float32) for _ in range(NBUF)]
        + [pltpu.SemaphoreType.DMA for _ in range(2 * NBUF)]
    ),
)
def _sc_gather(tok_hbm, codes_hbm, out_hbm, idx_v, *bufs_sems):
    bufs = bufs_sems[:NBUF]
    sem_g = bufs_sems[NBUF : 2 * NBUF]
    sem_w = bufs_sems[2 * NBUF :]
    wid = lax.axis_index("s") * NC + lax.axis_index("c")
    base = wid * BPW

    pltpu.sync_copy(tok_hbm.at[pl.ds(base, BPW)], idx_v)

    def start_gather(c, b):
        pltpu.async_copy(
            codes_hbm.at[idx_v.at[pl.ds(c * CH, CH)]], bufs[b], sem_g[b]
        )

    def wait_gather(c, b):
        pltpu.make_async_copy(
            codes_hbm.at[idx_v.at[pl.ds(c * CH, CH)]], bufs[b], sem_g[b]
        ).wait()

    def start_write(c, b):
        pltpu.async_copy(bufs[b], out_hbm.at[pl.ds(base + c * CH, CH)], sem_w[b])

    def wait_write(c, b):
        pltpu.make_async_copy(
            bufs[b], out_hbm.at[pl.ds(base + c * CH, CH)], sem_w[b]
        ).wait()

    for b in range(LA):
        start_gather(b, b)

    def group(gi, carry):
        c0 = gi * NBUF
        for b in range(NBUF):
            c = c0 + b
            wait_gather(c, b)
            start_write(c, b)
            nxt = c + LA
            sb = (b + LA) % NBUF

            @pl.when(nxt < NCH)
            def _():
                @pl.when(nxt >= NBUF)
                def _():
                    wait_write(nxt - NBUF, sb)

                start_gather(nxt, sb)

        return carry

    lax.fori_loop(0, NCH // NBUF, group, 0)

    for b in range(NBUF):
        wait_write(NCH - NBUF + b, b)


def kernel(tokens, codes):
    idx = tokens.reshape(-1).astype(jnp.int32)
    out = _sc_gather(idx, codes)
    return out.reshape(tokens.shape + (D,))
